# Initial kernel scaffold; baseline (speedup 1.0000x reference)
import jax
import jax.numpy as jnp
from jax.experimental import pallas as pl


def kernel(coord, feat, offset, W, gamma, beta):
    raise NotImplementedError("write your pallas kernel here")


# speedup vs baseline: 35.5302x; 35.5302x over previous
import functools

import jax
import jax.numpy as jnp
from jax import lax
from jax.experimental import pallas as pl
from jax.experimental.pallas import tpu as pltpu

NSAMP = 2048
EPSV = 1e-5
LANES = 128


def _stats_kernel(feat_ref, w_ref, s1_ref, s2_ref):
    k = pl.program_id(0)
    h = lax.dot_general(feat_ref[...], w_ref[...], (((1,), (1,)), ((), ())),
                        preferred_element_type=jnp.float32)

    @pl.when(k == 0)
    def _():
        s1_ref[...] = jnp.zeros_like(s1_ref)
        s2_ref[...] = jnp.zeros_like(s2_ref)

    s1_ref[...] += jnp.sum(h, axis=0, keepdims=True)
    s2_ref[...] += jnp.sum(h * h, axis=0, keepdims=True)


def _fps_kernel(*refs, num_clouds):
    B = num_clouds
    xyz = [refs[3 * b:3 * b + 3] for b in range(B)]
    reci_ref, recf_ref = refs[3 * B:3 * B + 2]
    dist = refs[3 * B + 2:3 * B + 2 + B]
    R, C = xyz[0][0].shape
    seg = R * C
    flat_io = (lax.broadcasted_iota(jnp.int32, (R, C), 0) * C
               + lax.broadcasted_iota(jnp.int32, (R, C), 1))
    lane_io = lax.broadcasted_iota(jnp.int32, (1, C), 1)

    def fetch(ref, r, c):
        rowv = ref[pl.ds(r, 1)]
        return jnp.sum(jnp.where(lane_io == c, rowv, 0.0))

    carry0 = []
    for b in range(B):
        dist[b][...] = jnp.full((R, C), 1e10, jnp.float32)
        carry0 += [jnp.int32(0),
                   jnp.sum(jnp.where(lane_io == 0, xyz[b][0][pl.ds(0, 1)], 0.0)),
                   jnp.sum(jnp.where(lane_io == 0, xyz[b][1][pl.ds(0, 1)], 0.0)),
                   jnp.sum(jnp.where(lane_io == 0, xyz[b][2][pl.ds(0, 1)], 0.0))]

    def it(i, carry):
        irow_i = jnp.zeros((1, C), jnp.int32)
        irow_f = jnp.zeros((1, C), jnp.float32)
        out = []
        for b in range(B):
            far, cx, cy, cz = carry[4 * b:4 * b + 4]
            irow_i = jnp.where(lane_io == b, far + jnp.int32(seg * b), irow_i)
            irow_f = jnp.where(lane_io == 3 * b, cx, irow_f)
            irow_f = jnp.where(lane_io == 3 * b + 1, cy, irow_f)
            irow_f = jnp.where(lane_io == 3 * b + 2, cz, irow_f)
            dx = xyz[b][0][...] - cx
            dy = xyz[b][1][...] - cy
            dz = xyz[b][2][...] - cz
            d = dx * dx + dy * dy + dz * dz
            dn = jnp.minimum(dist[b][...], d)
            dist[b][...] = dn
            m = jnp.max(dn)
            cand = jnp.where(dn == m, flat_io, jnp.int32(2147483647))
            nf = jnp.min(cand)
            r = nf // C
            c = nf - r * C
            out += [nf, fetch(xyz[b][0], r, c), fetch(xyz[b][1], r, c),
                    fetch(xyz[b][2], r, c)]
        reci_ref[pl.ds(i, 1), :] = irow_i
        recf_ref[pl.ds(i, 1), :] = irow_f
        return tuple(out)

    lax.fori_loop(0, NSAMP, it, tuple(carry0))


def _head_kernel(gat_ref, sub_ref, w_ref, s1_ref, s2_ref, g_ref, b_ref,
                 out_ref, *, n_total, cin):
    p = gat_ref[...]
    r = sub_ref[...]
    g = jnp.zeros((p.shape[0], cin), jnp.float32)
    for q in range(p.shape[1] // cin):
        g = jnp.where(r == q, p[:, q * cin:(q + 1) * cin], g)
    n = jnp.float32(n_total)
    mean = s1_ref[...] / n
    var = s2_ref[...] / n - mean * mean
    scale = g_ref[...] * lax.rsqrt(var + EPSV)
    shift = b_ref[...] - mean * scale
    h = lax.dot_general(g, w_ref[...], (((1,), (1,)), ((), ())),
                        preferred_element_type=jnp.float32)
    out_ref[...] = jnp.maximum(h * scale + shift, 0.0)


def _sc_gather(table, idx2d, out_rows, shift):
    from jax.experimental.pallas import tpu_sc as plsc

    info = plsc.get_sparse_core_info()
    nc, ns = info.num_cores, info.num_subcores
    nw = nc * ns
    rows_w = out_rows // nw
    chunks = rows_w // 128
    assert chunks * 128 == rows_w
    mesh = plsc.VectorSubcoreMesh(core_axis_name="c", subcore_axis_name="s")

    @functools.partial(
        pl.kernel,
        out_type=jax.ShapeDtypeStruct((out_rows, 128), jnp.float32),
        mesh=mesh,
        scratch_types=[
            pltpu.VMEM((chunks, 128), jnp.int32),
            pltpu.VMEM((rows_w, 128), jnp.float32),
            pltpu.SemaphoreType.DMA,
        ],
    )
    def gather_kernel(table_hbm, idx_hbm, out_hbm, idx_v, rows_v, sem):
        wid = lax.axis_index("s") * nc + lax.axis_index("c")
        pltpu.sync_copy(idx_hbm.at[pl.ds(wid * chunks, chunks)], idx_v)
        sh = jnp.full((16,), shift, jnp.int32)
        for j in range(chunks):
            for k in range(8):
                v = idx_v[j, pl.ds(k * 16, 16)]
                idx_v[j, pl.ds(k * 16, 16)] = lax.shift_right_logical(v, sh)
        for j in range(chunks):
            pltpu.async_copy(
                table_hbm.at[idx_v.at[j]],
                rows_v.at[pl.ds(j * 128, 128)],
                sem,
            ).wait()
        pltpu.sync_copy(rows_v, out_hbm.at[pl.ds(wid * rows_w, rows_w)])

    return gather_kernel(table, idx2d)


def kernel(coord, feat, offset, W, gamma, beta):
    N, _ = coord.shape
    B = offset.shape[0]
    Cin = feat.shape[1]
    Cout = W.shape[0]
    seg = N // B
    R = seg // LANES

    nblk = 32
    s1, s2 = pl.pallas_call(
        _stats_kernel,
        grid=(nblk,),
        in_specs=[
            pl.BlockSpec((N // nblk, Cin), lambda k: (k, 0)),
            pl.BlockSpec((Cout, Cin), lambda k: (0, 0)),
        ],
        out_specs=[
            pl.BlockSpec((1, Cout), lambda k: (0, 0)),
            pl.BlockSpec((1, Cout), lambda k: (0, 0)),
        ],
        out_shape=[
            jax.ShapeDtypeStruct((1, Cout), jnp.float32),
            jax.ShapeDtypeStruct((1, Cout), jnp.float32),
        ],
    )(feat, W)

    ct = coord.T.reshape(3, B, R, LANES)
    planes = [ct[k, b] for b in range(B) for k in range(3)]
    rec_i, rec_f = pl.pallas_call(
        functools.partial(_fps_kernel, num_clouds=B),
        out_shape=[
            jax.ShapeDtypeStruct((NSAMP, LANES), jnp.int32),
            jax.ShapeDtypeStruct((NSAMP, LANES), jnp.float32),
        ],
        scratch_shapes=[pltpu.VMEM((R, LANES), jnp.float32) for _ in range(B)],
    )(*planes)

    idx_global = rec_i[:, :B].T.reshape(B * NSAMP // LANES, LANES)
    new_coords = (
        rec_f[:, :3 * B].reshape(NSAMP, B, 3).transpose(1, 0, 2).reshape(B * NSAMP, 3)
    )

    pack = LANES // Cin
    shift = pack.bit_length() - 1
    gat = _sc_gather(feat.reshape(-1, LANES), idx_global, B * NSAMP, shift)
    sub = (idx_global.reshape(B * NSAMP, 1) & (pack - 1)).astype(jnp.int32)

    new_feats = pl.pallas_call(
        functools.partial(_head_kernel, n_total=N, cin=Cin),
        out_shape=jax.ShapeDtypeStruct((B * NSAMP, Cout), jnp.float32),
    )(gat, sub, W, s1, s2, gamma.reshape(1, Cout), beta.reshape(1, Cout))

    new_offsets = (jnp.arange(B, dtype=jnp.int32) + 1) * NSAMP
    return (new_coords, new_feats, new_offsets)


# speedup vs baseline: 58.6443x; 1.6505x over previous
import functools

import jax
import jax.numpy as jnp
from jax import lax
from jax.experimental import pallas as pl
from jax.experimental.pallas import tpu as pltpu

NSAMP = 2048
EPSV = 1e-5
LANES = 128


def _stats_kernel(feat_ref, w_ref, s1_ref, s2_ref):
    k = pl.program_id(0)
    h = lax.dot_general(feat_ref[...], w_ref[...], (((1,), (1,)), ((), ())),
                        preferred_element_type=jnp.float32)

    @pl.when(k == 0)
    def _():
        s1_ref[...] = jnp.zeros_like(s1_ref)
        s2_ref[...] = jnp.zeros_like(s2_ref)

    s1_ref[...] += jnp.sum(h, axis=0, keepdims=True)
    s2_ref[...] += jnp.sum(h * h, axis=0, keepdims=True)


def _fps_kernel(*refs, num_clouds):
    B = num_clouds
    xyz = [refs[3 * b:3 * b + 3] for b in range(B)]
    reci_ref, recf_ref = refs[3 * B:3 * B + 2]
    dist = refs[3 * B + 2:3 * B + 2 + B]
    R, C = xyz[0][0].shape
    seg = R * C
    flat_io = (lax.broadcasted_iota(jnp.int32, (R, C), 0) * C
               + lax.broadcasted_iota(jnp.int32, (R, C), 1))
    lane_io = lax.broadcasted_iota(jnp.int32, (1, C), 1)

    def fetch(ref, r, c):
        rowv = ref[pl.ds(r, 1)]
        return jnp.sum(jnp.where(lane_io == c, rowv, 0.0))

    carry0 = []
    for b in range(B):
        dist[b][...] = jnp.full((R, C), 1e10, jnp.float32)
        carry0 += [jnp.int32(0),
                   jnp.sum(jnp.where(lane_io == 0, xyz[b][0][pl.ds(0, 1)], 0.0)),
                   jnp.sum(jnp.where(lane_io == 0, xyz[b][1][pl.ds(0, 1)], 0.0)),
                   jnp.sum(jnp.where(lane_io == 0, xyz[b][2][pl.ds(0, 1)], 0.0))]

    CH = min(64, R)
    nch = R // CH
    lflat_io = flat_io[:CH]

    def it(i, carry):
        irow_i = jnp.zeros((1, C), jnp.int32)
        irow_f = jnp.zeros((1, C), jnp.float32)
        all_mts = []
        for b in range(B):
            far, cx, cy, cz = carry[4 * b:4 * b + 4]
            irow_i = jnp.where(lane_io == b, far + jnp.int32(seg * b), irow_i)
            irow_f = jnp.where(lane_io == 3 * b, cx, irow_f)
            irow_f = jnp.where(lane_io == 3 * b + 1, cy, irow_f)
            irow_f = jnp.where(lane_io == 3 * b + 2, cz, irow_f)
            mts = []
            for t in range(nch):
                sl = pl.ds(t * CH, CH)
                dx = xyz[b][0][sl] - cx
                dy = xyz[b][1][sl] - cy
                dz = xyz[b][2][sl] - cz
                d = dx * dx + dy * dy + dz * dz
                dn = jnp.minimum(dist[b][sl], d)
                dist[b][sl] = dn
                mts.append(jnp.max(dn))
            all_mts.append(mts)
        reci_ref[pl.ds(i, 1), :] = irow_i
        recf_ref[pl.ds(i, 1), :] = irow_f
        out = []
        for b in range(B):
            mts = all_mts[b]
            m = mts[0]
            for t in range(1, nch):
                m = jnp.maximum(m, mts[t])
            ksel = jnp.int32(nch)
            for t in reversed(range(nch)):
                ksel = jnp.where(mts[t] == m, jnp.int32(t), ksel)
            dnk = dist[b][pl.ds(ksel * CH, CH)]
            cand = jnp.where(dnk == m, lflat_io, jnp.int32(2147483647))
            nf = ksel * (CH * C) + jnp.min(cand)
            r = nf // C
            c = nf - r * C
            out += [nf, fetch(xyz[b][0], r, c), fetch(xyz[b][1], r, c),
                    fetch(xyz[b][2], r, c)]
        return tuple(out)

    lax.fori_loop(0, NSAMP, it, tuple(carry0))


def _head_kernel(gat_ref, sub_ref, w_ref, s1_ref, s2_ref, g_ref, b_ref,
                 out_ref, *, n_total, cin):
    p = gat_ref[...]
    r = sub_ref[...]
    g = jnp.zeros((p.shape[0], cin), jnp.float32)
    for q in range(p.shape[1] // cin):
        g = jnp.where(r == q, p[:, q * cin:(q + 1) * cin], g)
    n = jnp.float32(n_total)
    mean = s1_ref[...] / n
    var = s2_ref[...] / n - mean * mean
    scale = g_ref[...] * lax.rsqrt(var + EPSV)
    shift = b_ref[...] - mean * scale
    h = lax.dot_general(g, w_ref[...], (((1,), (1,)), ((), ())),
                        preferred_element_type=jnp.float32)
    out_ref[...] = jnp.maximum(h * scale + shift, 0.0)


def _sc_gather(table, idx2d, out_rows, shift):
    from jax.experimental.pallas import tpu_sc as plsc

    info = plsc.get_sparse_core_info()
    nc, ns = info.num_cores, info.num_subcores
    nw = nc * ns
    rows_w = out_rows // nw
    chunks = rows_w // 128
    assert chunks * 128 == rows_w
    mesh = plsc.VectorSubcoreMesh(core_axis_name="c", subcore_axis_name="s")

    @functools.partial(
        pl.kernel,
        out_type=jax.ShapeDtypeStruct((out_rows, 128), jnp.float32),
        mesh=mesh,
        scratch_types=[
            pltpu.VMEM((chunks, 128), jnp.int32),
            pltpu.VMEM((rows_w, 128), jnp.float32),
            pltpu.SemaphoreType.DMA,
        ],
    )
    def gather_kernel(table_hbm, idx_hbm, out_hbm, idx_v, rows_v, sem):
        wid = lax.axis_index("s") * nc + lax.axis_index("c")
        pltpu.sync_copy(idx_hbm.at[pl.ds(wid * chunks, chunks)], idx_v)
        sh = jnp.full((16,), shift, jnp.int32)
        for j in range(chunks):
            for k in range(8):
                v = idx_v[j, pl.ds(k * 16, 16)]
                idx_v[j, pl.ds(k * 16, 16)] = lax.shift_right_logical(v, sh)
        for j in range(chunks):
            pltpu.async_copy(
                table_hbm.at[idx_v.at[j]],
                rows_v.at[pl.ds(j * 128, 128)],
                sem,
            ).wait()
        pltpu.sync_copy(rows_v, out_hbm.at[pl.ds(wid * rows_w, rows_w)])

    return gather_kernel(table, idx2d)


def kernel(coord, feat, offset, W, gamma, beta):
    N, _ = coord.shape
    B = offset.shape[0]
    Cin = feat.shape[1]
    Cout = W.shape[0]
    seg = N // B
    R = seg // LANES

    nblk = 32
    s1, s2 = pl.pallas_call(
        _stats_kernel,
        grid=(nblk,),
        in_specs=[
            pl.BlockSpec((N // nblk, Cin), lambda k: (k, 0)),
            pl.BlockSpec((Cout, Cin), lambda k: (0, 0)),
        ],
        out_specs=[
            pl.BlockSpec((1, Cout), lambda k: (0, 0)),
            pl.BlockSpec((1, Cout), lambda k: (0, 0)),
        ],
        out_shape=[
            jax.ShapeDtypeStruct((1, Cout), jnp.float32),
            jax.ShapeDtypeStruct((1, Cout), jnp.float32),
        ],
    )(feat, W)

    ct = coord.T.reshape(3, B, R, LANES)
    planes = [ct[k, b] for b in range(B) for k in range(3)]
    rec_i, rec_f = pl.pallas_call(
        functools.partial(_fps_kernel, num_clouds=B),
        out_shape=[
            jax.ShapeDtypeStruct((NSAMP, LANES), jnp.int32),
            jax.ShapeDtypeStruct((NSAMP, LANES), jnp.float32),
        ],
        scratch_shapes=[pltpu.VMEM((R, LANES), jnp.float32) for _ in range(B)],
    )(*planes)

    idx_global = rec_i[:, :B].T.reshape(B * NSAMP // LANES, LANES)
    new_coords = (
        rec_f[:, :3 * B].reshape(NSAMP, B, 3).transpose(1, 0, 2).reshape(B * NSAMP, 3)
    )

    pack = LANES // Cin
    shift = pack.bit_length() - 1
    gat = _sc_gather(feat.reshape(-1, LANES), idx_global, B * NSAMP, shift)
    sub = (idx_global.reshape(B * NSAMP, 1) & (pack - 1)).astype(jnp.int32)

    new_feats = pl.pallas_call(
        functools.partial(_head_kernel, n_total=N, cin=Cin),
        out_shape=jax.ShapeDtypeStruct((B * NSAMP, Cout), jnp.float32),
    )(gat, sub, W, s1, s2, gamma.reshape(1, Cout), beta.reshape(1, Cout))

    new_offsets = (jnp.arange(B, dtype=jnp.int32) + 1) * NSAMP
    return (new_coords, new_feats, new_offsets)


# speedup vs baseline: 76.9541x; 1.3122x over previous
import functools

import jax
import jax.numpy as jnp
from jax import lax
from jax.experimental import pallas as pl
from jax.experimental.pallas import tpu as pltpu

NSAMP = 2048
EPSV = 1e-5
LANES = 128


def _stats_kernel(feat_ref, w_ref, s1_ref, s2_ref):
    k = pl.program_id(0)
    h = lax.dot_general(feat_ref[...], w_ref[...], (((1,), (1,)), ((), ())),
                        preferred_element_type=jnp.float32)

    @pl.when(k == 0)
    def _():
        s1_ref[...] = jnp.zeros_like(s1_ref)
        s2_ref[...] = jnp.zeros_like(s2_ref)

    s1_ref[...] += jnp.sum(h, axis=0, keepdims=True)
    s2_ref[...] += jnp.sum(h * h, axis=0, keepdims=True)


def _fps_kernel(*refs, num_clouds):
    B = num_clouds
    xyz = [refs[3 * b:3 * b + 3] for b in range(B)]
    reci_ref, recf_ref = refs[3 * B:3 * B + 2]
    dist = refs[3 * B + 2:3 * B + 2 + B]
    R, C = xyz[0][0].shape
    seg = R * C
    flat_io = (lax.broadcasted_iota(jnp.int32, (R, C), 0) * C
               + lax.broadcasted_iota(jnp.int32, (R, C), 1))
    lane_io = lax.broadcasted_iota(jnp.int32, (1, C), 1)

    def fetch(ref, r, c):
        rowv = ref[pl.ds(r, 1)]
        return jnp.sum(jnp.where(lane_io == c, rowv, 0.0))

    carry0 = []
    for b in range(B):
        dist[b][...] = jnp.full((R, C), 1e10, jnp.float32)
        carry0 += [jnp.int32(0),
                   jnp.sum(jnp.where(lane_io == 0, xyz[b][0][pl.ds(0, 1)], 0.0)),
                   jnp.sum(jnp.where(lane_io == 0, xyz[b][1][pl.ds(0, 1)], 0.0)),
                   jnp.sum(jnp.where(lane_io == 0, xyz[b][2][pl.ds(0, 1)], 0.0))]

    CH = min(64, R)
    nch = R // CH
    lflat_io = flat_io[:CH]

    def it(i, carry):
        irow_i = jnp.zeros((1, C), jnp.int32)
        irow_f = jnp.zeros((1, C), jnp.float32)
        all_mts = []
        for b in range(B):
            far, cx, cy, cz = carry[4 * b:4 * b + 4]
            irow_i = jnp.where(lane_io == b, far + jnp.int32(seg * b), irow_i)
            irow_f = jnp.where(lane_io == 3 * b, cx, irow_f)
            irow_f = jnp.where(lane_io == 3 * b + 1, cy, irow_f)
            irow_f = jnp.where(lane_io == 3 * b + 2, cz, irow_f)
            mts = []
            for t in range(nch):
                sl = pl.ds(t * CH, CH)
                dx = xyz[b][0][sl] - cx
                dy = xyz[b][1][sl] - cy
                dz = xyz[b][2][sl] - cz
                d = dx * dx + dy * dy + dz * dz
                dn = jnp.minimum(dist[b][sl], d)
                dist[b][sl] = dn
                mts.append(jnp.max(dn))
            all_mts.append(mts)
        reci_ref[pl.ds(i, 1), :] = irow_i
        recf_ref[pl.ds(i, 1), :] = irow_f
        out = []
        for b in range(B):
            mts = all_mts[b]
            m = mts[0]
            for t in range(1, nch):
                m = jnp.maximum(m, mts[t])
            ksel = jnp.int32(nch)
            for t in reversed(range(nch)):
                ksel = jnp.where(mts[t] == m, jnp.int32(t), ksel)
            dnk = dist[b][pl.ds(ksel * CH, CH)]
            cand = jnp.where(dnk == m, lflat_io, jnp.int32(2147483647))
            nf = ksel * (CH * C) + jnp.min(cand)
            r = nf // C
            c = nf - r * C
            out += [nf, fetch(xyz[b][0], r, c), fetch(xyz[b][1], r, c),
                    fetch(xyz[b][2], r, c)]
        return tuple(out)

    def it16(j, carry):
        for u in range(16):
            carry = it(16 * j + u, carry)
        return carry

    lax.fori_loop(0, NSAMP // 16, it16, tuple(carry0))


def _head_kernel(gat_ref, sub_ref, w_ref, s1_ref, s2_ref, g_ref, b_ref,
                 out_ref, *, n_total, cin):
    p = gat_ref[...]
    r = sub_ref[...]
    g = jnp.zeros((p.shape[0], cin), jnp.float32)
    for q in range(p.shape[1] // cin):
        g = jnp.where(r == q, p[:, q * cin:(q + 1) * cin], g)
    n = jnp.float32(n_total)
    mean = s1_ref[...] / n
    var = s2_ref[...] / n - mean * mean
    scale = g_ref[...] * lax.rsqrt(var + EPSV)
    shift = b_ref[...] - mean * scale
    h = lax.dot_general(g, w_ref[...], (((1,), (1,)), ((), ())),
                        preferred_element_type=jnp.float32)
    out_ref[...] = jnp.maximum(h * scale + shift, 0.0)


def _sc_gather(table, idx2d, out_rows, shift):
    from jax.experimental.pallas import tpu_sc as plsc

    info = plsc.get_sparse_core_info()
    nc, ns = info.num_cores, info.num_subcores
    nw = nc * ns
    rows_w = out_rows // nw
    chunks = rows_w // 128
    assert chunks * 128 == rows_w
    mesh = plsc.VectorSubcoreMesh(core_axis_name="c", subcore_axis_name="s")

    @functools.partial(
        pl.kernel,
        out_type=jax.ShapeDtypeStruct((out_rows, 128), jnp.float32),
        mesh=mesh,
        scratch_types=[
            pltpu.VMEM((chunks, 128), jnp.int32),
            pltpu.VMEM((rows_w, 128), jnp.float32),
            pltpu.SemaphoreType.DMA,
        ],
    )
    def gather_kernel(table_hbm, idx_hbm, out_hbm, idx_v, rows_v, sem):
        wid = lax.axis_index("s") * nc + lax.axis_index("c")
        pltpu.sync_copy(idx_hbm.at[pl.ds(wid * chunks, chunks)], idx_v)
        sh = jnp.full((16,), shift, jnp.int32)
        for j in range(chunks):
            for k in range(8):
                v = idx_v[j, pl.ds(k * 16, 16)]
                idx_v[j, pl.ds(k * 16, 16)] = lax.shift_right_logical(v, sh)
        for j in range(chunks):
            pltpu.async_copy(
                table_hbm.at[idx_v.at[j]],
                rows_v.at[pl.ds(j * 128, 128)],
                sem,
            ).wait()
        pltpu.sync_copy(rows_v, out_hbm.at[pl.ds(wid * rows_w, rows_w)])

    return gather_kernel(table, idx2d)


def kernel(coord, feat, offset, W, gamma, beta):
    N, _ = coord.shape
    B = offset.shape[0]
    Cin = feat.shape[1]
    Cout = W.shape[0]
    seg = N // B
    R = seg // LANES

    nblk = 32
    s1, s2 = pl.pallas_call(
        _stats_kernel,
        grid=(nblk,),
        in_specs=[
            pl.BlockSpec((N // nblk, Cin), lambda k: (k, 0)),
            pl.BlockSpec((Cout, Cin), lambda k: (0, 0)),
        ],
        out_specs=[
            pl.BlockSpec((1, Cout), lambda k: (0, 0)),
            pl.BlockSpec((1, Cout), lambda k: (0, 0)),
        ],
        out_shape=[
            jax.ShapeDtypeStruct((1, Cout), jnp.float32),
            jax.ShapeDtypeStruct((1, Cout), jnp.float32),
        ],
    )(feat, W)

    ct = coord.T.reshape(3, B, R, LANES)
    planes = [ct[k, b] for b in range(B) for k in range(3)]
    rec_i, rec_f = pl.pallas_call(
        functools.partial(_fps_kernel, num_clouds=B),
        out_shape=[
            jax.ShapeDtypeStruct((NSAMP, LANES), jnp.int32),
            jax.ShapeDtypeStruct((NSAMP, LANES), jnp.float32),
        ],
        scratch_shapes=[pltpu.VMEM((R, LANES), jnp.float32) for _ in range(B)],
    )(*planes)

    idx_global = rec_i[:, :B].T.reshape(B * NSAMP // LANES, LANES)
    new_coords = (
        rec_f[:, :3 * B].reshape(NSAMP, B, 3).transpose(1, 0, 2).reshape(B * NSAMP, 3)
    )

    pack = LANES // Cin
    shift = pack.bit_length() - 1
    gat = _sc_gather(feat.reshape(-1, LANES), idx_global, B * NSAMP, shift)
    sub = (idx_global.reshape(B * NSAMP, 1) & (pack - 1)).astype(jnp.int32)

    new_feats = pl.pallas_call(
        functools.partial(_head_kernel, n_total=N, cin=Cin),
        out_shape=jax.ShapeDtypeStruct((B * NSAMP, Cout), jnp.float32),
    )(gat, sub, W, s1, s2, gamma.reshape(1, Cout), beta.reshape(1, Cout))

    new_offsets = (jnp.arange(B, dtype=jnp.int32) + 1) * NSAMP
    return (new_coords, new_feats, new_offsets)


# speedup vs baseline: 77.5668x; 1.0080x over previous
import functools

import jax
import jax.numpy as jnp
from jax import lax
from jax.experimental import pallas as pl
from jax.experimental.pallas import tpu as pltpu

NSAMP = 2048
EPSV = 1e-5
LANES = 128


def _stats_kernel(feat_ref, w_ref, s1_ref, s2_ref):
    k = pl.program_id(0)
    h = lax.dot_general(feat_ref[...], w_ref[...], (((1,), (1,)), ((), ())),
                        preferred_element_type=jnp.float32)

    @pl.when(k == 0)
    def _():
        s1_ref[...] = jnp.zeros_like(s1_ref)
        s2_ref[...] = jnp.zeros_like(s2_ref)

    s1_ref[...] += jnp.sum(h, axis=0, keepdims=True)
    s2_ref[...] += jnp.sum(h * h, axis=0, keepdims=True)


def _fps_kernel(*refs, num_clouds):
    B = num_clouds
    xyz = [refs[3 * b:3 * b + 3] for b in range(B)]
    reci_ref, recf_ref = refs[3 * B:3 * B + 2]
    dist = refs[3 * B + 2:3 * B + 2 + B]
    R, C = xyz[0][0].shape
    seg = R * C
    flat_io = (lax.broadcasted_iota(jnp.int32, (R, C), 0) * C
               + lax.broadcasted_iota(jnp.int32, (R, C), 1))
    lane_io = lax.broadcasted_iota(jnp.int32, (1, C), 1)

    def fetch(ref, r, c):
        rowv = ref[pl.ds(r, 1)]
        return jnp.sum(jnp.where(lane_io == c, rowv, 0.0))

    carry0 = []
    for b in range(B):
        dist[b][...] = jnp.full((R, C), 1e10, jnp.float32)
        carry0 += [jnp.int32(0),
                   jnp.sum(jnp.where(lane_io == 0, xyz[b][0][pl.ds(0, 1)], 0.0)),
                   jnp.sum(jnp.where(lane_io == 0, xyz[b][1][pl.ds(0, 1)], 0.0)),
                   jnp.sum(jnp.where(lane_io == 0, xyz[b][2][pl.ds(0, 1)], 0.0))]

    CH = min(128, R)
    nch = R // CH
    lflat_io = flat_io[:CH]

    def it(i, carry):
        irow_i = jnp.zeros((1, C), jnp.int32)
        irow_f = jnp.zeros((1, C), jnp.float32)
        all_mts = []
        for b in range(B):
            far, cx, cy, cz = carry[4 * b:4 * b + 4]
            irow_i = jnp.where(lane_io == b, far + jnp.int32(seg * b), irow_i)
            irow_f = jnp.where(lane_io == 3 * b, cx, irow_f)
            irow_f = jnp.where(lane_io == 3 * b + 1, cy, irow_f)
            irow_f = jnp.where(lane_io == 3 * b + 2, cz, irow_f)
            mts = []
            for t in range(nch):
                sl = pl.ds(t * CH, CH)
                dx = xyz[b][0][sl] - cx
                dy = xyz[b][1][sl] - cy
                dz = xyz[b][2][sl] - cz
                d = (dx * dx + dz * dz) + dy * dy
                dn = jnp.minimum(dist[b][sl], d)
                dist[b][sl] = dn
                mts.append(jnp.max(dn))
            all_mts.append(mts)
        reci_ref[pl.ds(i, 1), :] = irow_i
        recf_ref[pl.ds(i, 1), :] = irow_f
        out = []
        for b in range(B):
            mts = all_mts[b]
            m = mts[0]
            for t in range(1, nch):
                m = jnp.maximum(m, mts[t])
            ksel = jnp.int32(nch)
            for t in reversed(range(nch)):
                ksel = jnp.where(mts[t] == m, jnp.int32(t), ksel)
            dnk = dist[b][pl.ds(ksel * CH, CH)]
            cand = jnp.where(dnk == m, lflat_io, jnp.int32(2147483647))
            nf = ksel * (CH * C) + jnp.min(cand)
            r = nf // C
            c = nf - r * C
            out += [nf, fetch(xyz[b][0], r, c), fetch(xyz[b][1], r, c),
                    fetch(xyz[b][2], r, c)]
        return tuple(out)

    def it16(j, carry):
        for u in range(16):
            carry = it(16 * j + u, carry)
        return carry

    lax.fori_loop(0, NSAMP // 16, it16, tuple(carry0))


def _head_kernel(gat_ref, sub_ref, w_ref, s1_ref, s2_ref, g_ref, b_ref,
                 out_ref, *, n_total, cin):
    p = gat_ref[...]
    r = sub_ref[...]
    g = jnp.zeros((p.shape[0], cin), jnp.float32)
    for q in range(p.shape[1] // cin):
        g = jnp.where(r == q, p[:, q * cin:(q + 1) * cin], g)
    n = jnp.float32(n_total)
    mean = s1_ref[...] / n
    var = s2_ref[...] / n - mean * mean
    scale = g_ref[...] * lax.rsqrt(var + EPSV)
    shift = b_ref[...] - mean * scale
    h = lax.dot_general(g, w_ref[...], (((1,), (1,)), ((), ())),
                        preferred_element_type=jnp.float32)
    out_ref[...] = jnp.maximum(h * scale + shift, 0.0)


def _sc_gather(table, idx2d, out_rows, shift):
    from jax.experimental.pallas import tpu_sc as plsc

    info = plsc.get_sparse_core_info()
    nc, ns = info.num_cores, info.num_subcores
    nw = nc * ns
    rows_w = out_rows // nw
    chunks = rows_w // 128
    assert chunks * 128 == rows_w
    mesh = plsc.VectorSubcoreMesh(core_axis_name="c", subcore_axis_name="s")

    @functools.partial(
        pl.kernel,
        out_type=jax.ShapeDtypeStruct((out_rows, 128), jnp.float32),
        mesh=mesh,
        scratch_types=[
            pltpu.VMEM((chunks, 128), jnp.int32),
            pltpu.VMEM((rows_w, 128), jnp.float32),
            pltpu.SemaphoreType.DMA,
        ],
    )
    def gather_kernel(table_hbm, idx_hbm, out_hbm, idx_v, rows_v, sem):
        wid = lax.axis_index("s") * nc + lax.axis_index("c")
        pltpu.sync_copy(idx_hbm.at[pl.ds(wid * chunks, chunks)], idx_v)
        sh = jnp.full((16,), shift, jnp.int32)
        for j in range(chunks):
            for k in range(8):
                v = idx_v[j, pl.ds(k * 16, 16)]
                idx_v[j, pl.ds(k * 16, 16)] = lax.shift_right_logical(v, sh)
        for j in range(chunks):
            pltpu.async_copy(
                table_hbm.at[idx_v.at[j]],
                rows_v.at[pl.ds(j * 128, 128)],
                sem,
            ).wait()
        pltpu.sync_copy(rows_v, out_hbm.at[pl.ds(wid * rows_w, rows_w)])

    return gather_kernel(table, idx2d)


def kernel(coord, feat, offset, W, gamma, beta):
    N, _ = coord.shape
    B = offset.shape[0]
    Cin = feat.shape[1]
    Cout = W.shape[0]
    seg = N // B
    R = seg // LANES

    nblk = 32
    s1, s2 = pl.pallas_call(
        _stats_kernel,
        grid=(nblk,),
        in_specs=[
            pl.BlockSpec((N // nblk, Cin), lambda k: (k, 0)),
            pl.BlockSpec((Cout, Cin), lambda k: (0, 0)),
        ],
        out_specs=[
            pl.BlockSpec((1, Cout), lambda k: (0, 0)),
            pl.BlockSpec((1, Cout), lambda k: (0, 0)),
        ],
        out_shape=[
            jax.ShapeDtypeStruct((1, Cout), jnp.float32),
            jax.ShapeDtypeStruct((1, Cout), jnp.float32),
        ],
    )(feat, W)

    ct = coord.T.reshape(3, B, R, LANES)
    planes = [ct[k, b] for b in range(B) for k in range(3)]
    rec_i, rec_f = pl.pallas_call(
        functools.partial(_fps_kernel, num_clouds=B),
        out_shape=[
            jax.ShapeDtypeStruct((NSAMP, LANES), jnp.int32),
            jax.ShapeDtypeStruct((NSAMP, LANES), jnp.float32),
        ],
        scratch_shapes=[pltpu.VMEM((R, LANES), jnp.float32) for _ in range(B)],
    )(*planes)

    idx_global = rec_i[:, :B].T.reshape(B * NSAMP // LANES, LANES)
    new_coords = (
        rec_f[:, :3 * B].reshape(NSAMP, B, 3).transpose(1, 0, 2).reshape(B * NSAMP, 3)
    )

    pack = LANES // Cin
    shift = pack.bit_length() - 1
    gat = _sc_gather(feat.reshape(-1, LANES), idx_global, B * NSAMP, shift)
    sub = (idx_global.reshape(B * NSAMP, 1) & (pack - 1)).astype(jnp.int32)

    new_feats = pl.pallas_call(
        functools.partial(_head_kernel, n_total=N, cin=Cin),
        out_shape=jax.ShapeDtypeStruct((B * NSAMP, Cout), jnp.float32),
    )(gat, sub, W, s1, s2, gamma.reshape(1, Cout), beta.reshape(1, Cout))

    new_offsets = (jnp.arange(B, dtype=jnp.int32) + 1) * NSAMP
    return (new_coords, new_feats, new_offsets)
